# Initial kernel scaffold; baseline (speedup 1.0000x reference)
import jax
import jax.numpy as jnp
from jax.experimental import pallas as pl


def kernel(inputs, w1, w2, w3, rweights, selected_experts):
    raise NotImplementedError("write your pallas kernel here")


# speedup vs baseline: 1.4256x; 1.4256x over previous
import functools

import jax
import jax.numpy as jnp
from jax import lax
from jax.experimental import pallas as pl
from jax.experimental.pallas import tpu as pltpu
from jax.experimental.pallas import tpu_sc as plsc

_T = 2048
_DIM = 4096
_HID = 14336
_E = 8
_K = 2

_TM = 256
_NT = (_T * _K) // _TM + _E
_NROWS = _NT * _TM
_TH = 256
_NH = _HID // _TH

_NC = 2
_NS = 16
_NW = _NC * _NS
_L = 16


def _routing_metadata(rweights, selected_experts):
    e_flat = selected_experts.reshape(-1).astype(jnp.int32)
    w_flat = rweights.reshape(-1)
    t_flat = jnp.repeat(jnp.arange(_T, dtype=jnp.int32), _K)

    onehot = e_flat[:, None] == jnp.arange(_E, dtype=jnp.int32)[None, :]
    counts = jnp.sum(onehot.astype(jnp.int32), axis=0)
    ranks = jnp.cumsum(onehot.astype(jnp.int32), axis=0) - 1
    r = jnp.sum(jnp.where(onehot, ranks, 0), axis=1)
    padded = ((counts + _TM - 1) // _TM) * _TM
    ends = jnp.cumsum(padded)
    base = ends - padded
    dest = base[e_flat] + r

    row_token = jnp.zeros((_NROWS,), jnp.int32).at[dest].set(t_flat)
    row_w = jnp.zeros((_NROWS,), jnp.float32).at[dest].set(w_flat)
    tile_expert = jnp.clip(
        jnp.searchsorted(ends, jnp.arange(_NT, dtype=jnp.int32) * _TM,
                         side="right"),
        0, _E - 1).astype(jnp.int32)
    d = dest.reshape(_T, _K)
    return row_token, row_w, tile_expert, d[:, 0], d[:, 1]


_G_CH = 16
_G_RPW = _NROWS // _NW

@functools.cache
def _sc_mesh():
    return plsc.VectorSubcoreMesh(core_axis_name="c", subcore_axis_name="s",
                                  num_cores=_NC, num_subcores=_NS)


@functools.cache
def _sc_gather_rows():
    @functools.partial(
        pl.kernel,
        out_type=jax.ShapeDtypeStruct((_NROWS, _DIM), jnp.float32),
        mesh=_sc_mesh(),
        scratch_types=[
            pltpu.VMEM((_G_CH,), jnp.int32),
            pltpu.VMEM((_G_CH, _DIM), jnp.float32),
            pltpu.SemaphoreType.DMA,
        ],
    )
    def gather_rows(inp_hbm, idx_hbm, out_hbm, idx_v, rows_v, sem):
        wid = lax.axis_index("s") * _NC + lax.axis_index("c")
        worker_base = wid * _G_RPW

        def chunk(c, carry):
            off = worker_base + c * _G_CH
            pltpu.sync_copy(idx_hbm.at[pl.ds(off, _G_CH)], idx_v)
            pltpu.async_copy(inp_hbm.at[idx_v], rows_v, sem).wait()
            pltpu.sync_copy(rows_v, out_hbm.at[pl.ds(off, _G_CH)])
            return carry

        lax.fori_loop(0, _G_RPW // _G_CH, chunk, 0)

    return gather_rows


def _ffn_body(te_ref, xg_ref, w1_ref, w3_ref, w2_ref, rw_ref, out_ref):
    j = pl.program_id(1)
    x = xg_ref[...]
    a = lax.dot_general(x, w1_ref[0], (((1,), (1,)), ((), ())),
                        preferred_element_type=jnp.float32)
    b = lax.dot_general(x, w3_ref[0], (((1,), (1,)), ((), ())),
                        preferred_element_type=jnp.float32)
    h = a * lax.logistic(a) * b
    y = lax.dot_general(h, w2_ref[0], (((1,), (0,)), ((), ())),
                        preferred_element_type=jnp.float32)

    @pl.when(j == 0)
    def _():
        out_ref[...] = jnp.zeros_like(out_ref)

    out_ref[...] += y

    @pl.when(j == _NH - 1)
    def _():
        out_ref[...] *= rw_ref[...]


def _tc_ffn(tile_expert, xg, w1, w3, w2, row_w):
    grid_spec = pltpu.PrefetchScalarGridSpec(
        num_scalar_prefetch=1,
        grid=(_NT, _NH),
        in_specs=[
            pl.BlockSpec((_TM, _DIM), lambda i, j, te: (i, 0)),
            pl.BlockSpec((1, _TH, _DIM), lambda i, j, te: (te[i], j, 0)),
            pl.BlockSpec((1, _TH, _DIM), lambda i, j, te: (te[i], j, 0)),
            pl.BlockSpec((1, _TH, _DIM), lambda i, j, te: (te[i], j, 0)),
            pl.BlockSpec((_TM, 1), lambda i, j, te: (i, 0)),
        ],
        out_specs=pl.BlockSpec((_TM, _DIM), lambda i, j, te: (i, 0)),
    )
    return pl.pallas_call(
        _ffn_body,
        grid_spec=grid_spec,
        out_shape=jax.ShapeDtypeStruct((_NROWS, _DIM), jnp.float32),
        compiler_params=pltpu.CompilerParams(
            dimension_semantics=("arbitrary", "arbitrary")),
    )(tile_expert, xg, w1, w3, w2, row_w.reshape(_NROWS, 1))


_C_CH = 8
_C_TPW = _T // _NW


@functools.cache
def _sc_combine_rows():
    @functools.partial(
        pl.kernel,
        out_type=jax.ShapeDtypeStruct((_T, _DIM), jnp.float32),
        mesh=_sc_mesh(),
        scratch_types=[
            pltpu.VMEM((_C_CH,), jnp.int32),
            pltpu.VMEM((_C_CH,), jnp.int32),
            pltpu.VMEM((_C_CH, _DIM), jnp.float32),
            pltpu.VMEM((_C_CH, _DIM), jnp.float32),
            pltpu.SemaphoreType.DMA,
            pltpu.SemaphoreType.DMA,
        ],
    )
    def combine_rows(yg_hbm, d0_hbm, d1_hbm, out_hbm,
                     i0_v, i1_v, a_v, b_v, s0, s1):
        wid = lax.axis_index("s") * _NC + lax.axis_index("c")
        worker_base = wid * _C_TPW

        def chunk(c, carry):
            off = worker_base + c * _C_CH
            pltpu.sync_copy(d0_hbm.at[pl.ds(off, _C_CH)], i0_v)
            pltpu.sync_copy(d1_hbm.at[pl.ds(off, _C_CH)], i1_v)
            cp0 = pltpu.async_copy(yg_hbm.at[i0_v], a_v, s0)
            cp1 = pltpu.async_copy(yg_hbm.at[i1_v], b_v, s1)
            cp0.wait()
            cp1.wait()
            for row in range(_C_CH):
                def add_group(g, c2, row=row):
                    sl = pl.ds(g * _L, _L)
                    a_v[row, sl] = a_v[row, sl] + b_v[row, sl]
                    return c2
                lax.fori_loop(0, _DIM // _L, add_group, 0)
            pltpu.sync_copy(a_v, out_hbm.at[pl.ds(off, _C_CH)])
            return carry

        lax.fori_loop(0, _C_TPW // _C_CH, chunk, 0)

    return combine_rows


def kernel(inputs, w1, w2, w3, rweights, selected_experts):
    row_token, row_w, tile_expert, d0, d1 = _routing_metadata(
        rweights, selected_experts)
    xg = _sc_gather_rows()(inputs, row_token)
    yg = _tc_ffn(tile_expert, xg, w1, w3, w2, row_w)
    return _sc_combine_rows()(yg, d0, d1)
